# baseline (speedup 1.0000x reference)
import functools

import jax
import jax.numpy as jnp
from jax import lax
from jax.experimental import pallas as pl
from jax.experimental.pallas import tpu as pltpu
from jax.experimental.pallas import tpu_sc as plsc

B = 16384
D = 256
DW = D // 2
NNE = 10
L = 16

_info = plsc.get_sparse_core_info()
NC = _info.num_cores
NS = _info.num_subcores
NW = NC * NS
BPW = B // NW

CH = 8
GI = CH * NNE
NIT = BPW // CH
SG = 128
NSG = BPW // SG
NRING = 4
NBI = BPW * NNE
NCI = NBI + BPW

_HIMASK = jnp.int32(-65536)

_mesh = plsc.VectorSubcoreMesh(core_axis_name="c", subcore_axis_name="s")


@functools.partial(
    pl.kernel,
    mesh=_mesh,
    out_type=(
        jax.ShapeDtypeStruct((B, DW), jnp.int32),
        jax.ShapeDtypeStruct((B, D), jnp.float32),
    ),
    scratch_types=[
        pltpu.VMEM((NCI,), jnp.int32),
        pltpu.VMEM((SG, DW), jnp.int32),
        pltpu.VMEM((SG, DW), jnp.int32),
        pltpu.VMEM((GI, DW), jnp.int32),
        pltpu.VMEM((GI, DW), jnp.int32),
        pltpu.VMEM((GI, DW), jnp.int32),
        pltpu.VMEM((GI, DW), jnp.int32),
        pltpu.VMEM((CH, D), jnp.float32),
        pltpu.VMEM((CH, D), jnp.float32),
        pltpu.SemaphoreType.DMA,
        pltpu.SemaphoreType.DMA,
        pltpu.SemaphoreType.DMA,
        pltpu.SemaphoreType.DMA,
        pltpu.SemaphoreType.DMA,
        pltpu.SemaphoreType.DMA,
        pltpu.SemaphoreType.DMA,
        pltpu.SemaphoreType.DMA,
    ],
)
def _sc_gather_mean(neigh_hbm, nodes_hbm, table_hbm, self_out, sum_out,
                    cidx_v, srows0, srows1, gb0, gb1, gb2, gb3,
                    ob0, ob1, gs0, gs1, gs2, gs3, os0, os1, as0, as1):
    wid = lax.axis_index("s") * NC + lax.axis_index("c")
    base = wid * BPW

    pltpu.sync_copy(neigh_hbm.at[wid], cidx_v.at[pl.ds(0, NBI)])
    pltpu.sync_copy(nodes_hbm.at[wid], cidx_v.at[pl.ds(NBI, BPW)])

    sbufs = (srows0, srows1)
    gbufs = (gb0, gb1, gb2, gb3)
    gsems = (gs0, gs1, gs2, gs3)
    obufs = (ob0, ob1)
    osems = (os0, os1)
    asems = (as0, as1)

    def sidx(g):
        return cidx_v.at[pl.ds(NBI + g * SG, SG)]

    pltpu.make_async_copy(table_hbm.at[sidx(0)], srows0, gs0).start()
    for g in range(NSG):
        b = g % 2
        if g + 1 < NSG:
            nxt = (g + 1) % 2
            if g >= 1:
                pltpu.make_async_copy(
                    sbufs[nxt], self_out.at[pl.ds(base, SG)],
                    asems[nxt]).wait()
            pltpu.make_async_copy(
                table_hbm.at[sidx(g + 1)], sbufs[nxt], gsems[nxt]).start()
        pltpu.make_async_copy(
            table_hbm.at[sidx(g)], sbufs[b], gsems[b]).wait()
        pltpu.make_async_copy(
            sbufs[b], self_out.at[pl.ds(base + g * SG, SG)], asems[b]).start()

    for b in range(NRING - 1):
        pltpu.make_async_copy(
            table_hbm.at[cidx_v.at[pl.ds(b * GI, GI)]],
            gbufs[b], gsems[b]).start()

    def ring_body(g, _):
        for b in range(NRING):
            it = g * NRING + b
            ob = b % 2
            pltpu.make_async_copy(
                table_hbm.at[cidx_v.at[pl.ds(it * GI, GI)]],
                gbufs[b], gsems[b]).wait()

            if b >= 2:
                pltpu.make_async_copy(
                    obufs[ob], sum_out.at[pl.ds(base, CH)],
                    osems[ob]).wait()
            else:
                @pl.when(g >= 1)
                def _():
                    pltpu.make_async_copy(
                        obufs[ob], sum_out.at[pl.ds(base, CH)],
                        osems[ob]).wait()

            @pl.when(it + NRING - 1 < NIT)
            def _():
                pltpu.make_async_copy(
                    table_hbm.at[
                        cidx_v.at[pl.ds((it + NRING - 1) * GI, GI)]],
                    gbufs[(b + NRING - 1) % NRING],
                    gsems[(b + NRING - 1) % NRING]).start()

            def node_body(n, _):
                for c in range(DW // L):
                    col = pl.ds(c * L, L)
                    w = gbufs[b][n * NNE, col]
                    lo = lax.bitcast_convert_type(w << 16, jnp.float32)
                    hi = lax.bitcast_convert_type(w & _HIMASK, jnp.float32)
                    for j in range(1, NNE):
                        w = gbufs[b][n * NNE + j, col]
                        lo = lo + lax.bitcast_convert_type(
                            w << 16, jnp.float32)
                        hi = hi + lax.bitcast_convert_type(
                            w & _HIMASK, jnp.float32)
                    obufs[ob][n, pl.ds(c * L, L)] = lo
                    obufs[ob][n, pl.ds(DW + c * L, L)] = hi
                return 0

            lax.fori_loop(0, CH, node_body, 0)

            pltpu.make_async_copy(
                obufs[ob],
                sum_out.at[pl.ds(base + it * CH, CH)],
                osems[ob]).start()
        return 0

    lax.fori_loop(0, NIT // NRING, ring_body, 0)

    for ob in range(2):
        pltpu.make_async_copy(
            obufs[ob], sum_out.at[pl.ds(base, CH)], osems[ob]).wait()
        pltpu.make_async_copy(
            sbufs[ob], self_out.at[pl.ds(base, SG)], asems[ob]).wait()


def _rne_bf16_bits(x):
    xi = lax.bitcast_convert_type(x, jnp.int32)
    lsb = (xi >> 16) & jnp.int32(1)
    return ((xi + jnp.int32(0x7FFF) + lsb) >> 16) & jnp.int32(0xFFFF)


def _mm_body(s_ref, m_ref, w1a_ref, w1b_ref, w2_ref, o_ref):
    w = s_ref[...]
    lo = lax.bitcast_convert_type(w << 16, jnp.float32)
    hi = lax.bitcast_convert_type(w & jnp.int32(-65536), jnp.float32)
    acc = jnp.dot(lo, w1a_ref[...], preferred_element_type=jnp.float32)
    acc += jnp.dot(hi, w1b_ref[...], preferred_element_type=jnp.float32)
    acc += jnp.dot(m_ref[...], w2_ref[...], preferred_element_type=jnp.float32)
    o_ref[...] = jnp.maximum(acc, 0.0)


_BM = 2048


@jax.jit
def kernel(feat_table, nodes, neigh_idx, weight):
    neigh_r = neigh_idx.astype(jnp.int32).reshape(NW, NBI)
    nodes_r = nodes.astype(jnp.int32).reshape(NW, BPW)

    lo_bits = _rne_bf16_bits(feat_table[:, :DW])
    hi_bits = _rne_bf16_bits(feat_table[:, DW:])
    tbits = lo_bits | (hi_bits << 16)

    self_b, sum_f = _sc_gather_mean(neigh_r, nodes_r, tbits)

    w2 = weight[D:] * (1.0 / NNE)

    out = pl.pallas_call(
        _mm_body,
        grid=(B // _BM,),
        in_specs=[
            pl.BlockSpec((_BM, DW), lambda i: (i, 0)),
            pl.BlockSpec((_BM, D), lambda i: (i, 0)),
            pl.BlockSpec((DW, D), lambda i: (0, 0)),
            pl.BlockSpec((DW, D), lambda i: (0, 0)),
            pl.BlockSpec((D, D), lambda i: (0, 0)),
        ],
        out_specs=pl.BlockSpec((_BM, D), lambda i: (i, 0)),
        out_shape=jax.ShapeDtypeStruct((B, D), jnp.float32),
    )(self_b, sum_f, weight[:DW], weight[DW:D], w2)
    return out


# speedup vs baseline: 1.0156x; 1.0156x over previous
import functools

import jax
import jax.numpy as jnp
from jax import lax
from jax.experimental import pallas as pl
from jax.experimental.pallas import tpu as pltpu
from jax.experimental.pallas import tpu_sc as plsc

B = 16384
D = 256
DW = D // 2
NNE = 10
L = 16

_info = plsc.get_sparse_core_info()
NC = _info.num_cores
NS = _info.num_subcores
NW = NC * NS
BPW = B // NW

CH = 8
GI = CH * NNE
NIT = BPW // CH
SG = 128
NSG = BPW // SG
NRING = 4
NBI = BPW * NNE
NCI = NBI + BPW

_HIMASK = jnp.int32(-65536)

_mesh = plsc.VectorSubcoreMesh(core_axis_name="c", subcore_axis_name="s")


@functools.partial(
    pl.kernel,
    mesh=_mesh,
    out_type=(
        jax.ShapeDtypeStruct((B, DW), jnp.int32),
        jax.ShapeDtypeStruct((B, D), jnp.float32),
    ),
    scratch_types=[
        pltpu.VMEM((NCI,), jnp.int32),
        pltpu.VMEM((SG, DW), jnp.int32),
        pltpu.VMEM((SG, DW), jnp.int32),
        pltpu.VMEM((GI, DW), jnp.int32),
        pltpu.VMEM((GI, DW), jnp.int32),
        pltpu.VMEM((GI, DW), jnp.int32),
        pltpu.VMEM((GI, DW), jnp.int32),
        pltpu.VMEM((CH, D), jnp.float32),
        pltpu.VMEM((CH, D), jnp.float32),
        pltpu.SemaphoreType.DMA,
        pltpu.SemaphoreType.DMA,
        pltpu.SemaphoreType.DMA,
        pltpu.SemaphoreType.DMA,
        pltpu.SemaphoreType.DMA,
        pltpu.SemaphoreType.DMA,
        pltpu.SemaphoreType.DMA,
        pltpu.SemaphoreType.DMA,
    ],
)
def _sc_gather_mean(cidx_hbm, table_hbm, self_out, sum_out,
                    cidx_v, srows0, srows1, gb0, gb1, gb2, gb3,
                    ob0, ob1, gs0, gs1, gs2, gs3, os0, os1, as0, as1):
    wid = lax.axis_index("s") * NC + lax.axis_index("c")
    base = wid * BPW

    pltpu.sync_copy(cidx_hbm.at[wid], cidx_v)

    sbufs = (srows0, srows1)
    gbufs = (gb0, gb1, gb2, gb3)
    gsems = (gs0, gs1, gs2, gs3)
    obufs = (ob0, ob1)
    osems = (os0, os1)
    asems = (as0, as1)

    def sidx(g):
        return cidx_v.at[pl.ds(NBI + g * SG, SG)]

    pltpu.make_async_copy(table_hbm.at[sidx(0)], srows0, gs0).start()
    for g in range(NSG):
        b = g % 2
        if g + 1 < NSG:
            nxt = (g + 1) % 2
            if g >= 1:
                pltpu.make_async_copy(
                    sbufs[nxt], self_out.at[pl.ds(base, SG)],
                    asems[nxt]).wait()
            pltpu.make_async_copy(
                table_hbm.at[sidx(g + 1)], sbufs[nxt], gsems[nxt]).start()
        pltpu.make_async_copy(
            table_hbm.at[sidx(g)], sbufs[b], gsems[b]).wait()
        pltpu.make_async_copy(
            sbufs[b], self_out.at[pl.ds(base + g * SG, SG)], asems[b]).start()

    for b in range(NRING - 1):
        pltpu.make_async_copy(
            table_hbm.at[cidx_v.at[pl.ds(b * GI, GI)]],
            gbufs[b], gsems[b]).start()

    def ring_body(g, _):
        for b in range(NRING):
            it = g * NRING + b
            ob = b % 2
            pltpu.make_async_copy(
                table_hbm.at[cidx_v.at[pl.ds(it * GI, GI)]],
                gbufs[b], gsems[b]).wait()

            if b >= 2:
                pltpu.make_async_copy(
                    obufs[ob], sum_out.at[pl.ds(base, CH)],
                    osems[ob]).wait()
            else:
                @pl.when(g >= 1)
                def _():
                    pltpu.make_async_copy(
                        obufs[ob], sum_out.at[pl.ds(base, CH)],
                        osems[ob]).wait()

            def node_body(n, _):
                for c in range(DW // L):
                    col = pl.ds(c * L, L)
                    w = gbufs[b][n * NNE, col]
                    lo = lax.bitcast_convert_type(w << 16, jnp.float32)
                    hi = lax.bitcast_convert_type(w & _HIMASK, jnp.float32)
                    for j in range(1, NNE):
                        w = gbufs[b][n * NNE + j, col]
                        lo = lo + lax.bitcast_convert_type(
                            w << 16, jnp.float32)
                        hi = hi + lax.bitcast_convert_type(
                            w & _HIMASK, jnp.float32)
                    obufs[ob][n, pl.ds(c * L, L)] = lo
                    obufs[ob][n, pl.ds(DW + c * L, L)] = hi
                return 0

            lax.fori_loop(0, CH, node_body, 0)

            @pl.when(it + NRING - 1 < NIT)
            def _():
                pltpu.make_async_copy(
                    table_hbm.at[
                        cidx_v.at[pl.ds((it + NRING - 1) * GI, GI)]],
                    gbufs[(b + NRING - 1) % NRING],
                    gsems[(b + NRING - 1) % NRING]).start()

            pltpu.make_async_copy(
                obufs[ob],
                sum_out.at[pl.ds(base + it * CH, CH)],
                osems[ob]).start()
        return 0

    lax.fori_loop(0, NIT // NRING, ring_body, 0)

    for ob in range(2):
        pltpu.make_async_copy(
            obufs[ob], sum_out.at[pl.ds(base, CH)], osems[ob]).wait()
        pltpu.make_async_copy(
            sbufs[ob], self_out.at[pl.ds(base, SG)], asems[ob]).wait()


def _rne_bf16_bits(x):
    xi = lax.bitcast_convert_type(x, jnp.int32)
    lsb = (xi >> 16) & jnp.int32(1)
    return ((xi + jnp.int32(0x7FFF) + lsb) >> 16) & jnp.int32(0xFFFF)


def _mm_body(s_ref, m_ref, w1a_ref, w1b_ref, w2_ref, o_ref):
    w = s_ref[...]
    lo = lax.bitcast_convert_type(w << 16, jnp.float32)
    hi = lax.bitcast_convert_type(w & jnp.int32(-65536), jnp.float32)
    acc = jnp.dot(lo, w1a_ref[...], preferred_element_type=jnp.float32)
    acc += jnp.dot(hi, w1b_ref[...], preferred_element_type=jnp.float32)
    acc += jnp.dot(m_ref[...], w2_ref[...], preferred_element_type=jnp.float32)
    o_ref[...] = jnp.maximum(acc, 0.0)


_BM = 2048


@jax.jit
def kernel(feat_table, nodes, neigh_idx, weight):
    cidx = jnp.concatenate(
        [neigh_idx.astype(jnp.int32).reshape(NW, NBI),
         nodes.astype(jnp.int32).reshape(NW, BPW)], axis=1)

    lo_bits = _rne_bf16_bits(feat_table[:, :DW])
    hi_bits = _rne_bf16_bits(feat_table[:, DW:])
    tbits = lo_bits | (hi_bits << 16)

    self_b, sum_f = _sc_gather_mean(cidx, tbits)

    w2 = weight[D:] * (1.0 / NNE)

    out = pl.pallas_call(
        _mm_body,
        grid=(B // _BM,),
        in_specs=[
            pl.BlockSpec((_BM, DW), lambda i: (i, 0)),
            pl.BlockSpec((_BM, D), lambda i: (i, 0)),
            pl.BlockSpec((DW, D), lambda i: (0, 0)),
            pl.BlockSpec((DW, D), lambda i: (0, 0)),
            pl.BlockSpec((D, D), lambda i: (0, 0)),
        ],
        out_specs=pl.BlockSpec((_BM, D), lambda i: (i, 0)),
        out_shape=jax.ShapeDtypeStruct((B, D), jnp.float32),
    )(self_b, sum_f, weight[:DW], weight[DW:D], w2)
    return out
